# baseline (speedup 1.0000x reference)
import functools

import jax
import jax.numpy as jnp
from jax import lax
from jax.experimental import pallas as pl
from jax.experimental.pallas import tpu as pltpu
from jax.experimental.pallas import tpu_sc as plsc

_NC = 2
_NS = 16
_K = 80
_CW = 16
_SEC = 25


@functools.lru_cache(maxsize=None)
def _build(N, E, D, R):
    NW = _NC * _NS
    assert E % (NW * _K) == 0
    CPW = E // (NW * _K)
    ZR = 125
    assert (N // _NS) % ZR == 0
    BN = 1000
    assert N % BN == 0 and N % _NS == 0

    def s1a_body(h_ref, w_ref, table_ref):
        table_ref[...] = jnp.dot(h_ref[...], w_ref[...],
                                 preferred_element_type=jnp.float32)

    stage1a = pl.pallas_call(
        s1a_body,
        grid=(N // BN,),
        in_specs=[
            pl.BlockSpec((BN, D), lambda i: (i, 0)),
            pl.BlockSpec((D, D), lambda i: (0, 0)),
        ],
        out_specs=pl.BlockSpec((BN, D), lambda i: (i, 0)),
        out_shape=jax.ShapeDtypeStruct((N, D), jnp.float32),
    )

    def s1b_body(h_ref, lw_ref, lb_ref, lm_ref):
        lm_ref[...] = (
            jnp.dot(h_ref[...], lw_ref[...],
                    preferred_element_type=jnp.float32)
            + lb_ref[...]
        )

    stage1b = pl.pallas_call(
        s1b_body,
        grid=(N // BN,),
        in_specs=[
            pl.BlockSpec((BN, D), lambda i: (i, 0)),
            pl.BlockSpec((D, D), lambda i: (0, 0)),
            pl.BlockSpec((1, D), lambda i: (0, 0)),
        ],
        out_specs=pl.BlockSpec((BN, D), lambda i: (i, 0)),
        out_shape=jax.ShapeDtypeStruct((N, D), jnp.float32),
    )

    return (stage1a, stage1b, _build_sc(N, E, D, R),
            _build_combine(N, E, D, R))


@functools.lru_cache(maxsize=None)
def _build_counts(N, E, D, R):
    NW = _NC * _NS
    CPW = E // (NW * _K)
    ZR = 25

    mesh = plsc.VectorSubcoreMesh(core_axis_name="c", subcore_axis_name="s",
                                  num_cores=_NC, num_subcores=_NS)

    @functools.partial(
        pl.kernel,
        out_type=jax.ShapeDtypeStruct((_NC, N, _CW), jnp.float32),
        mesh=mesh,
        scratch_types=[
            pltpu.VMEM((CPW, _K), jnp.int32),
            pltpu.VMEM((_K,), jnp.int32),
            pltpu.VMEM((_K,), jnp.int32),
            pltpu.VMEM((_K, _CW), jnp.float32),
            pltpu.VMEM((ZR, _CW), jnp.float32),
            pltpu.VMEM_SHARED((N, _CW), jnp.float32),
            pltpu.SemaphoreType.DMA,
            pltpu.SemaphoreType.DMA,
            pltpu.SemaphoreType.DMA,
        ],
        compiler_params=pltpu.CompilerParams(use_tc_tiling_on_sc=False),
    )
    def stage2c(pk_hbm, pcnt_hbm, pk_v, ds0, ds1, ones_v, zcnt_v, accc,
                cs0, cs1, z_sem):
        c = lax.axis_index("c")
        s = lax.axis_index("s")
        row0 = (c * _NS + s) * CPW

        def fill(i, _):
            zcnt_v[i, :] = jnp.zeros((_CW,), jnp.float32)
            return 0
        lax.fori_loop(0, ZR, fill, 0)

        def fill_ones(i, _):
            ones_v[i, :] = jnp.ones((_CW,), jnp.float32)
            return 0
        lax.fori_loop(0, _K, fill_ones, 0)

        nz = N // _NS // ZR

        def zero_acc(p, _):
            off = s * (N // _NS) + p * ZR
            pltpu.async_copy(zcnt_v, accc.at[pl.ds(off, ZR)], z_sem)
            return 0
        lax.fori_loop(0, nz, zero_acc, 0)

        pltpu.sync_copy(pk_hbm.at[pl.ds(row0, CPW)], pk_v)

        def drain_zero(p, _):
            off = s * (N // _NS) + p * ZR
            pltpu.make_async_copy(zcnt_v, accc.at[pl.ds(off, ZR)],
                                  z_sem).wait()
            return 0
        lax.fori_loop(0, nz, drain_zero, 0)

        def unpack_dst(i, db):
            def col(j, _):
                sl = pl.ds(j * 16, 16)
                db[sl] = pk_v[i, sl] & 0x3FFF
                return 0
            lax.fori_loop(0, _K // 16, col, 0)

        plsc.subcore_barrier()

        def half(cc, db, cs):
            @pl.when(cc >= 2)
            def _():
                pltpu.make_async_copy(ones_v, accc.at[db], cs).wait()

            @pl.when(cc < CPW)
            def _():
                unpack_dst(cc, db)
                pltpu.async_copy(ones_v, accc.at[db], cs, add=True)

        def pair(t, _):
            half(2 * t, ds0, cs0)
            half(2 * t + 1, ds1, cs1)
            return 0
        lax.fori_loop(0, (CPW + 2) // 2, pair, 0)

        pltpu.make_async_copy(ones_v, accc.at[ds0], cs0).wait()

        plsc.subcore_barrier()

        roff = s * (N // _NS)
        pltpu.sync_copy(accc.at[pl.ds(roff, N // _NS)],
                        pcnt_hbm.at[c, pl.ds(roff, N // _NS)])

    return stage2c


@functools.lru_cache(maxsize=None)
def _build_sc(N, E, D, R):
    NW = _NC * _NS
    CPW = E // (NW * _K)
    assert (CPW + 1) % 3 == 0
    assert CPW % _SEC == 0
    ZR = 25

    mesh = plsc.VectorSubcoreMesh(core_axis_name="c", subcore_axis_name="s",
                                  num_cores=_NC, num_subcores=_NS)

    @functools.partial(
        pl.kernel,
        out_type=[
            jax.ShapeDtypeStruct((_NC, N, D), jnp.float32),
            jax.ShapeDtypeStruct((_NC, N, _CW), jnp.float32),
        ],
        mesh=mesh,
        scratch_types=[
            pltpu.VMEM((_SEC, _K), jnp.int32),
            pltpu.VMEM((_K,), jnp.int32),
            pltpu.VMEM((_K,), jnp.int32),
            pltpu.VMEM((_K,), jnp.int32),
            pltpu.VMEM((_K,), jnp.int32),
            pltpu.VMEM((_K,), jnp.int32),
            pltpu.VMEM((_K,), jnp.int32),
            pltpu.VMEM((_K,), jnp.float32),
            pltpu.VMEM((_K,), jnp.float32),
            pltpu.VMEM((_K,), jnp.float32),
            pltpu.VMEM((_K, D), jnp.float32),
            pltpu.VMEM((_K, D), jnp.float32),
            pltpu.VMEM((_K, D), jnp.float32),
            pltpu.VMEM((16,), jnp.float32),
            pltpu.VMEM((ZR, D), jnp.float32),
            pltpu.VMEM((_K, _CW), jnp.float32),
            pltpu.VMEM((ZR, _CW), jnp.float32),
            pltpu.VMEM_SHARED((N, D), jnp.float32),
            pltpu.VMEM_SHARED((N, _CW), jnp.float32),
            pltpu.SemaphoreType.DMA,
            pltpu.SemaphoreType.DMA,
            pltpu.SemaphoreType.DMA,
            pltpu.SemaphoreType.DMA,
            pltpu.SemaphoreType.DMA,
            pltpu.SemaphoreType.DMA,
            pltpu.SemaphoreType.DMA,
            pltpu.SemaphoreType.DMA,
            pltpu.SemaphoreType.DMA,
            pltpu.SemaphoreType.DMA,
        ],
        compiler_params=pltpu.CompilerParams(use_tc_tiling_on_sc=False,
                                             needs_layout_passes=False),
    )
    def stage2(pk_hbm, table_hbm, wr_hbm, psum_hbm, pcnt_hbm,
               pk_v, gi0, ds0, gi1, ds1, gi2, ds2, al0, al1, al2,
               rows0, rows1, rows2, wr_v,
               zrow_v, ones_v, zcnt_v, acc, accc,
               g0, g1, g2, s0, s1, s2, cs0, cs1, cs2, z_sem):
        c = lax.axis_index("c")
        s = lax.axis_index("s")
        wid = c * _NS + s
        row0 = wid * CPW

        pltpu.sync_copy(wr_hbm, wr_v)
        w16 = wr_v[...]
        w16 = jnp.exp(w16 - jnp.max(w16, axis=0))
        wr_v[...] = w16 / jnp.sum(w16, axis=0)

        def fill_small(i, _):
            zcnt_v[i, :] = jnp.ones((_CW,), jnp.float32) * 0.0
            ones_v[i, :] = jnp.ones((_CW,), jnp.float32)
            return 0
        lax.fori_loop(0, ZR, fill_small, 0)

        def fill_ones2(i, _):
            ones_v[ZR + i, :] = jnp.ones((_CW,), jnp.float32)
            return 0
        lax.fori_loop(0, _K - ZR, fill_ones2, 0)

        def fill_rows(i, _):
            def fill_cols(j, _):
                zrow_v[i, pl.ds(j * 16, 16)] = jnp.zeros((16,), jnp.float32)
                return 0
            return lax.fori_loop(0, D // 16, fill_cols, 0)
        lax.fori_loop(0, ZR, fill_rows, 0)

        nz = N // _NS // ZR

        def zero_acc(p, _):
            off = s * (N // _NS) + p * ZR
            pltpu.async_copy(zrow_v, acc.at[pl.ds(off, ZR)], z_sem)
            pltpu.async_copy(zcnt_v, accc.at[pl.ds(off, ZR)], z_sem)
            return 0
        lax.fori_loop(0, nz, zero_acc, 0)

        pltpu.sync_copy(pk_hbm.at[pl.ds(row0, _SEC)], pk_v)

        def drain_zero(p, _):
            off = s * (N // _NS) + p * ZR
            pltpu.make_async_copy(zrow_v, acc.at[pl.ds(off, ZR)], z_sem).wait()
            pltpu.make_async_copy(zcnt_v, accc.at[pl.ds(off, ZR)],
                                  z_sem).wait()
            return 0
        lax.fori_loop(0, nz, drain_zero, 0)

        def unpack_into(i, gb, db, ab):
            def col(j, _):
                sl = pl.ds(j * 16, 16)
                w = pk_v[i, sl]
                db[sl] = w & 0x3FFF
                gb[sl] = lax.shift_right_logical(w, 14) & 0x3FFF
                typ = lax.shift_right_logical(w, 28)
                ab[sl] = plsc.load_gather(wr_v, [typ])
                return 0
            lax.fori_loop(0, _K // 16, col, 0)

        plsc.subcore_barrier()

        slots = ((rows0, gi0, ds0, al0, g0, s0, cs0),
                 (rows1, gi1, ds1, al1, g1, s1, cs1),
                 (rows2, gi2, ds2, al2, g2, s2, cs2))

        unpack_into(0, gi0, ds0, al0)
        pltpu.async_copy(table_hbm.at[gi0], rows0, g0)

        def scale_rows(rows, ab):
            def per_group(g, _):
                av = ab[pl.ds(g * 16, 16)]
                for lane in range(16):
                    a = av[lane]
                    e = g * 16 + lane
                    for q in range(D // 16):
                        sl = pl.ds(q * 16, 16)
                        rows[e, sl] = rows[e, sl] * a
                return 0
            lax.fori_loop(0, _K // 16, per_group, 0)

        def half(c, b):
            p_rows, p_gi, p_ds, p_al, p_g, p_s, p_cs = slots[(b + 2) % 3]
            x_rows, x_gi, x_ds, x_al, x_g, x_s, x_cs = slots[(b + 1) % 3]

            @pl.when(c >= 1)
            def _():
                pltpu.make_async_copy(table_hbm.at[p_gi], p_rows, p_g).wait()
                pltpu.async_copy(ones_v, accc.at[p_ds], p_cs, add=True)
                scale_rows(p_rows, p_al)
                pltpu.async_copy(p_rows, acc.at[p_ds], p_s, add=True)

            @pl.when(c >= 2)
            def _():
                pltpu.make_async_copy(x_rows, acc.at[x_ds], x_s).wait()
                pltpu.make_async_copy(ones_v, accc.at[x_ds], x_cs).wait()

            @pl.when(c + 1 <= CPW - 1)
            def _():
                nxt = c + 1

                @pl.when(nxt % _SEC == 0)
                def _():
                    pltpu.sync_copy(pk_hbm.at[pl.ds(row0 + nxt, _SEC)], pk_v)

                unpack_into(nxt % _SEC, x_gi, x_ds, x_al)
                pltpu.async_copy(table_hbm.at[x_gi], x_rows, x_g)

        def triple(t, _):
            half(3 * t, 0)
            half(3 * t + 1, 1)
            half(3 * t + 2, 2)
            return 0
        lax.fori_loop(0, (CPW + 1) // 3, triple, 0)

        f_rows, f_gi, f_ds, f_al, f_g, f_s, f_cs = slots[(CPW - 1) % 3]
        pltpu.make_async_copy(f_rows, acc.at[f_ds], f_s).wait()
        pltpu.make_async_copy(ones_v, accc.at[f_ds], f_cs).wait()

        plsc.subcore_barrier()

        roff = s * (N // _NS)
        pltpu.sync_copy(acc.at[pl.ds(roff, N // _NS)],
                        psum_hbm.at[c, pl.ds(roff, N // _NS)])
        pltpu.sync_copy(accc.at[pl.ds(roff, N // _NS)],
                        pcnt_hbm.at[c, pl.ds(roff, N // _NS)])

    return stage2


@functools.lru_cache(maxsize=None)
def _build_combine(N, E, D, R):
    BN = 1000

    def s3_body(psum_ref, pcnt_ref, lm_ref, h_ref, bias_ref, g_ref, out_ref):
        ssum = psum_ref[0] + psum_ref[1]
        cnt = (pcnt_ref[0] + pcnt_ref[1])[:, 0:1]
        mean = ssum / jnp.maximum(cnt, 1.0)
        node = jnp.where(cnt > 0, mean, h_ref[...])
        lm = lm_ref[...]
        logit = jnp.sum(lm * g_ref[0:1, :] + node * g_ref[1:2, :],
                        axis=1, keepdims=True)
        att = jax.nn.sigmoid(logit)
        node = node + bias_ref[...]
        out_ref[...] = node * att + lm * (1.0 - att)

    stage3 = pl.pallas_call(
        s3_body,
        grid=(N // BN,),
        in_specs=[
            pl.BlockSpec((_NC, BN, D), lambda i: (0, i, 0)),
            pl.BlockSpec((_NC, BN, _CW), lambda i: (0, i, 0)),
            pl.BlockSpec((BN, D), lambda i: (i, 0)),
            pl.BlockSpec((BN, D), lambda i: (i, 0)),
            pl.BlockSpec((1, D), lambda i: (0, 0)),
            pl.BlockSpec((2, D), lambda i: (0, 0)),
        ],
        out_specs=pl.BlockSpec((BN, D), lambda i: (i, 0)),
        out_shape=jax.ShapeDtypeStruct((N, D), jnp.float32),
    )

    return stage3


def kernel(h, edge_index, edge_type, W, loop_weight, loop_bias, bias_weight,
           weight_rel, gating_attention):
    N, D = h.shape
    E = edge_type.shape[0]
    R = weight_rel.shape[0]
    stage1a, stage1b, stage2, stage3 = _build(N, E, D, R)

    packed = lax.bitcast_convert_type(
        (edge_type.astype(jnp.uint32) << 28)
        | (edge_index[0].astype(jnp.uint32) << 14)
        | edge_index[1].astype(jnp.uint32), jnp.int32)
    packed2d = packed.reshape(E // _K, _K)

    table = stage1a(h, W)

    psum, pcnt = stage2(packed2d, table, weight_rel.reshape(R))

    loop_msg = stage1b(h, loop_weight, loop_bias.reshape(1, D))

    return stage3(psum, pcnt, loop_msg, h,
                  bias_weight.reshape(1, D), gating_attention.reshape(2, D))


# speedup vs baseline: 1.0138x; 1.0138x over previous
import functools

import jax
import jax.numpy as jnp
from jax import lax
from jax.experimental import pallas as pl
from jax.experimental.pallas import tpu as pltpu
from jax.experimental.pallas import tpu_sc as plsc

_NC = 2
_NS = 16
_K = 80
_CW = 16
_SEC = 25


@functools.lru_cache(maxsize=None)
def _build(N, E, D, R):
    NW = _NC * _NS
    assert E % (NW * _K) == 0
    CPW = E // (NW * _K)
    ZR = 125
    assert (N // _NS) % ZR == 0
    BN = 1000
    assert N % BN == 0 and N % _NS == 0

    def s1a_body(h_ref, w_ref, table_ref):
        table_ref[...] = jnp.dot(h_ref[...], w_ref[...],
                                 preferred_element_type=jnp.float32)

    stage1a = pl.pallas_call(
        s1a_body,
        grid=(N // BN,),
        in_specs=[
            pl.BlockSpec((BN, D), lambda i: (i, 0)),
            pl.BlockSpec((D, D), lambda i: (0, 0)),
        ],
        out_specs=pl.BlockSpec((BN, D), lambda i: (i, 0)),
        out_shape=jax.ShapeDtypeStruct((N, D), jnp.float32),
    )

    def s1b_body(h_ref, lw_ref, lb_ref, lm_ref):
        lm_ref[...] = (
            jnp.dot(h_ref[...], lw_ref[...],
                    preferred_element_type=jnp.float32)
            + lb_ref[...]
        )

    stage1b = pl.pallas_call(
        s1b_body,
        grid=(N // BN,),
        in_specs=[
            pl.BlockSpec((BN, D), lambda i: (i, 0)),
            pl.BlockSpec((D, D), lambda i: (0, 0)),
            pl.BlockSpec((1, D), lambda i: (0, 0)),
        ],
        out_specs=pl.BlockSpec((BN, D), lambda i: (i, 0)),
        out_shape=jax.ShapeDtypeStruct((N, D), jnp.float32),
    )

    return (stage1a, stage1b, _build_counts(N, E, D, R),
            _build_sc(N, E, D, R), _build_combine(N, E, D, R))


@functools.lru_cache(maxsize=None)
def _build_counts(N, E, D, R):
    NW = _NC * _NS
    CPW = E // (NW * _K)
    ZR = 25

    mesh = plsc.VectorSubcoreMesh(core_axis_name="c", subcore_axis_name="s",
                                  num_cores=_NC, num_subcores=_NS)

    @functools.partial(
        pl.kernel,
        out_type=jax.ShapeDtypeStruct((_NC, N, _CW), jnp.float32),
        mesh=mesh,
        scratch_types=[
            pltpu.VMEM((CPW, _K), jnp.int32),
            pltpu.VMEM((_K,), jnp.int32),
            pltpu.VMEM((_K,), jnp.int32),
            pltpu.VMEM((_K, _CW), jnp.float32),
            pltpu.VMEM((ZR, _CW), jnp.float32),
            pltpu.VMEM_SHARED((N, _CW), jnp.float32),
            pltpu.SemaphoreType.DMA,
            pltpu.SemaphoreType.DMA,
            pltpu.SemaphoreType.DMA,
        ],
        compiler_params=pltpu.CompilerParams(use_tc_tiling_on_sc=False),
    )
    def stage2c(pk_hbm, pcnt_hbm, pk_v, ds0, ds1, ones_v, zcnt_v, accc,
                cs0, cs1, z_sem):
        c = lax.axis_index("c")
        s = lax.axis_index("s")
        row0 = (c * _NS + s) * CPW

        def fill(i, _):
            zcnt_v[i, :] = jnp.zeros((_CW,), jnp.float32)
            return 0
        lax.fori_loop(0, ZR, fill, 0)

        def fill_ones(i, _):
            ones_v[i, :] = jnp.ones((_CW,), jnp.float32)
            return 0
        lax.fori_loop(0, _K, fill_ones, 0)

        nz = N // _NS // ZR

        def zero_acc(p, _):
            off = s * (N // _NS) + p * ZR
            pltpu.async_copy(zcnt_v, accc.at[pl.ds(off, ZR)], z_sem)
            return 0
        lax.fori_loop(0, nz, zero_acc, 0)

        pltpu.sync_copy(pk_hbm.at[pl.ds(row0, CPW)], pk_v)

        def drain_zero(p, _):
            off = s * (N // _NS) + p * ZR
            pltpu.make_async_copy(zcnt_v, accc.at[pl.ds(off, ZR)],
                                  z_sem).wait()
            return 0
        lax.fori_loop(0, nz, drain_zero, 0)

        def unpack_dst(i, db):
            def col(j, _):
                sl = pl.ds(j * 16, 16)
                db[sl] = pk_v[i, sl] & 0x3FFF
                return 0
            lax.fori_loop(0, _K // 16, col, 0)

        plsc.subcore_barrier()

        def half(cc, db, cs):
            @pl.when(cc >= 2)
            def _():
                pltpu.make_async_copy(ones_v, accc.at[db], cs).wait()

            @pl.when(cc < CPW)
            def _():
                unpack_dst(cc, db)
                pltpu.async_copy(ones_v, accc.at[db], cs, add=True)

        def pair(t, _):
            half(2 * t, ds0, cs0)
            half(2 * t + 1, ds1, cs1)
            return 0
        lax.fori_loop(0, (CPW + 2) // 2, pair, 0)

        pltpu.make_async_copy(ones_v, accc.at[ds0], cs0).wait()

        plsc.subcore_barrier()

        roff = s * (N // _NS)
        pltpu.sync_copy(accc.at[pl.ds(roff, N // _NS)],
                        pcnt_hbm.at[c, pl.ds(roff, N // _NS)])

    return stage2c


@functools.lru_cache(maxsize=None)
def _build_sc(N, E, D, R):
    NW = _NC * _NS
    CPW = E // (NW * _K)
    assert (CPW + 1) % 3 == 0
    assert CPW % _SEC == 0
    ZR = 25

    mesh = plsc.VectorSubcoreMesh(core_axis_name="c", subcore_axis_name="s",
                                  num_cores=_NC, num_subcores=_NS)

    @functools.partial(
        pl.kernel,
        out_type=jax.ShapeDtypeStruct((_NC, N, D), jnp.float32),
        mesh=mesh,
        scratch_types=[
            pltpu.VMEM((_SEC, _K), jnp.int32),
            pltpu.VMEM((_K,), jnp.int32),
            pltpu.VMEM((_K,), jnp.int32),
            pltpu.VMEM((_K,), jnp.int32),
            pltpu.VMEM((_K,), jnp.int32),
            pltpu.VMEM((_K,), jnp.int32),
            pltpu.VMEM((_K,), jnp.int32),
            pltpu.VMEM((_K,), jnp.float32),
            pltpu.VMEM((_K,), jnp.float32),
            pltpu.VMEM((_K,), jnp.float32),
            pltpu.VMEM((_K, D), jnp.float32),
            pltpu.VMEM((_K, D), jnp.float32),
            pltpu.VMEM((_K, D), jnp.float32),
            pltpu.VMEM((16,), jnp.float32),
            pltpu.VMEM((ZR, D), jnp.float32),
            pltpu.VMEM_SHARED((N, D), jnp.float32),
            pltpu.SemaphoreType.DMA,
            pltpu.SemaphoreType.DMA,
            pltpu.SemaphoreType.DMA,
            pltpu.SemaphoreType.DMA,
            pltpu.SemaphoreType.DMA,
            pltpu.SemaphoreType.DMA,
            pltpu.SemaphoreType.DMA,
        ],
        compiler_params=pltpu.CompilerParams(use_tc_tiling_on_sc=False,
                                             needs_layout_passes=False),
    )
    def stage2(pk_hbm, table_hbm, wr_hbm, psum_hbm,
               pk_v, gi0, ds0, gi1, ds1, gi2, ds2, al0, al1, al2,
               rows0, rows1, rows2, wr_v,
               zrow_v, acc, g0, g1, g2, s0, s1, s2, z_sem):
        c = lax.axis_index("c")
        s = lax.axis_index("s")
        wid = c * _NS + s
        row0 = wid * CPW

        pltpu.sync_copy(wr_hbm, wr_v)
        w16 = wr_v[...]
        w16 = jnp.exp(w16 - jnp.max(w16, axis=0))
        wr_v[...] = w16 / jnp.sum(w16, axis=0)

        def fill_rows(i, _):
            def fill_cols(j, _):
                zrow_v[i, pl.ds(j * 16, 16)] = jnp.zeros((16,), jnp.float32)
                return 0
            return lax.fori_loop(0, D // 16, fill_cols, 0)
        lax.fori_loop(0, ZR, fill_rows, 0)

        nz = N // _NS // ZR

        def zero_acc(p, _):
            off = s * (N // _NS) + p * ZR
            pltpu.async_copy(zrow_v, acc.at[pl.ds(off, ZR)], z_sem)
            return 0
        lax.fori_loop(0, nz, zero_acc, 0)

        pltpu.sync_copy(pk_hbm.at[pl.ds(row0, _SEC)], pk_v)

        def drain_zero(p, _):
            off = s * (N // _NS) + p * ZR
            pltpu.make_async_copy(zrow_v, acc.at[pl.ds(off, ZR)], z_sem).wait()
            return 0
        lax.fori_loop(0, nz, drain_zero, 0)

        def unpack_into(i, gb, db, ab):
            def col(j, _):
                sl = pl.ds(j * 16, 16)
                w = pk_v[i, sl]
                db[sl] = w & 0x3FFF
                gb[sl] = lax.shift_right_logical(w, 14) & 0x3FFF
                typ = lax.shift_right_logical(w, 28)
                ab[sl] = plsc.load_gather(wr_v, [typ])
                return 0
            lax.fori_loop(0, _K // 16, col, 0)

        plsc.subcore_barrier()

        slots = ((rows0, gi0, ds0, al0, g0, s0),
                 (rows1, gi1, ds1, al1, g1, s1),
                 (rows2, gi2, ds2, al2, g2, s2))

        unpack_into(0, gi0, ds0, al0)
        pltpu.async_copy(table_hbm.at[gi0], rows0, g0)

        def scale_rows(rows, ab):
            def per_group(g, _):
                av = ab[pl.ds(g * 16, 16)]
                for lane in range(16):
                    a = av[lane]
                    e = g * 16 + lane
                    for q in range(D // 16):
                        sl = pl.ds(q * 16, 16)
                        rows[e, sl] = rows[e, sl] * a
                return 0
            lax.fori_loop(0, _K // 16, per_group, 0)

        def half(c, b):
            p_rows, p_gi, p_ds, p_al, p_g, p_s = slots[(b + 2) % 3]
            x_rows, x_gi, x_ds, x_al, x_g, x_s = slots[(b + 1) % 3]

            @pl.when(c >= 1)
            def _():
                pltpu.make_async_copy(table_hbm.at[p_gi], p_rows, p_g).wait()
                scale_rows(p_rows, p_al)
                pltpu.async_copy(p_rows, acc.at[p_ds], p_s, add=True)

            @pl.when(c >= 2)
            def _():
                pltpu.make_async_copy(x_rows, acc.at[x_ds], x_s).wait()

            @pl.when(c + 1 <= CPW - 1)
            def _():
                nxt = c + 1

                @pl.when(nxt % _SEC == 0)
                def _():
                    pltpu.sync_copy(pk_hbm.at[pl.ds(row0 + nxt, _SEC)], pk_v)

                unpack_into(nxt % _SEC, x_gi, x_ds, x_al)
                pltpu.async_copy(table_hbm.at[x_gi], x_rows, x_g)

        def triple(t, _):
            half(3 * t, 0)
            half(3 * t + 1, 1)
            half(3 * t + 2, 2)
            return 0
        lax.fori_loop(0, (CPW + 1) // 3, triple, 0)

        f_rows, f_gi, f_ds, f_al, f_g, f_s = slots[(CPW - 1) % 3]
        pltpu.make_async_copy(f_rows, acc.at[f_ds], f_s).wait()

        plsc.subcore_barrier()

        roff = s * (N // _NS)
        pltpu.sync_copy(acc.at[pl.ds(roff, N // _NS)],
                        psum_hbm.at[c, pl.ds(roff, N // _NS)])

    return stage2


@functools.lru_cache(maxsize=None)
def _build_combine(N, E, D, R):
    BN = 1000

    def s3_body(psum_ref, pcnt_ref, lm_ref, h_ref, bias_ref, g_ref, out_ref):
        ssum = psum_ref[0] + psum_ref[1]
        cnt = (pcnt_ref[0] + pcnt_ref[1])[:, 0:1]
        mean = ssum / jnp.maximum(cnt, 1.0)
        node = jnp.where(cnt > 0, mean, h_ref[...])
        lm = lm_ref[...]
        logit = jnp.sum(lm * g_ref[0:1, :] + node * g_ref[1:2, :],
                        axis=1, keepdims=True)
        att = jax.nn.sigmoid(logit)
        node = node + bias_ref[...]
        out_ref[...] = node * att + lm * (1.0 - att)

    stage3 = pl.pallas_call(
        s3_body,
        grid=(N // BN,),
        in_specs=[
            pl.BlockSpec((_NC, BN, D), lambda i: (0, i, 0)),
            pl.BlockSpec((_NC, BN, _CW), lambda i: (0, i, 0)),
            pl.BlockSpec((BN, D), lambda i: (i, 0)),
            pl.BlockSpec((BN, D), lambda i: (i, 0)),
            pl.BlockSpec((1, D), lambda i: (0, 0)),
            pl.BlockSpec((2, D), lambda i: (0, 0)),
        ],
        out_specs=pl.BlockSpec((BN, D), lambda i: (i, 0)),
        out_shape=jax.ShapeDtypeStruct((N, D), jnp.float32),
    )

    return stage3


def kernel(h, edge_index, edge_type, W, loop_weight, loop_bias, bias_weight,
           weight_rel, gating_attention):
    N, D = h.shape
    E = edge_type.shape[0]
    R = weight_rel.shape[0]
    stage1a, stage1b, stage2c, stage2, stage3 = _build(N, E, D, R)

    packed = lax.bitcast_convert_type(
        (edge_type.astype(jnp.uint32) << 28)
        | (edge_index[0].astype(jnp.uint32) << 14)
        | edge_index[1].astype(jnp.uint32), jnp.int32)
    packed2d = packed.reshape(E // _K, _K)

    pcnt = stage2c(packed2d)

    table = stage1a(h, W)

    loop_msg = stage1b(h, loop_weight, loop_bias.reshape(1, D))

    psum = stage2(packed2d, table, weight_rel.reshape(R))

    return stage3(psum, pcnt, loop_msg, h,
                  bias_weight.reshape(1, D), gating_attention.reshape(2, D))
